# Initial kernel scaffold; baseline (speedup 1.0000x reference)
import jax
import jax.numpy as jnp
from jax.experimental import pallas as pl


def kernel(x, edge_index, W_self, W_neigh, b):
    raise NotImplementedError("write your pallas kernel here")


# speedup vs baseline: 5.8011x; 5.8011x over previous
import functools

import jax
import jax.numpy as jnp
from jax import lax
from jax.experimental import pallas as pl
from jax.experimental.pallas import tpu as pltpu
from jax.experimental.pallas import tpu_sc as plsc

_N = 10000
_E = 320000
_D = 128
_DH = _D // 2

_NC = 2
_NS = 16
_NW = _NC * _NS

_CHUNK = 128
_CHUNKS_PER_T = 160
_EDGES_PER_T = _CHUNK * _CHUNKS_PER_T
_E_PAD = _EDGES_PER_T * _NS
_N_PAD = 10112
_ROWS_PER_TILE = _N_PAD // _NS
_ACC_ROWS = _N
_LAST_ROWS = _ACC_ROWS - 15 * _ROWS_PER_TILE
_FULL_CHUNKS = 160
_LAST_CHUNKS = (_E - 15 * _EDGES_PER_T) // _CHUNK
_DEG_STAGE = 640
_DEG_HALF = _N_PAD // 2
_DEG_ROWS = _DEG_HALF + 64
_DEG_TILES = _DEG_HALF // _ROWS_PER_TILE


def _sc_aggregate_body(y0_hbm, y1_hbm, src_hbm, dst_hbm,
                       acc_out, deg_out,
                       src_v, dst_v, rows_v, ones_v, idxt_v, stage_acc, stage_deg,
                       acc_sh, deg_sh, gsem):
    c = lax.axis_index("c")
    s = lax.axis_index("s")
    base = s * _ROWS_PER_TILE

    pltpu.sync_copy(src_hbm.at[s], src_v)
    pltpu.sync_copy(dst_hbm.at[s], dst_v)

    z16 = jnp.zeros((16,), jnp.float32)

    def _zrow(i, carry):
        for k in range(_DH // 16):
            stage_acc[i, pl.ds(k * 16, 16)] = z16
        return carry

    lax.fori_loop(0, _ROWS_PER_TILE, _zrow, 0)

    def _zdeg(i, carry):
        stage_deg[pl.ds(i * 16, 16)] = z16
        return carry

    lax.fori_loop(0, _DEG_STAGE // 16, _zdeg, 0)

    @pl.when(s < 15)
    def _():
        pltpu.sync_copy(stage_acc, acc_sh.at[pl.ds(base, _ROWS_PER_TILE)])

    @pl.when(s == 15)
    def _():
        pltpu.sync_copy(stage_acc.at[pl.ds(0, _LAST_ROWS)],
                        acc_sh.at[pl.ds(15 * _ROWS_PER_TILE, _LAST_ROWS)])

    @pl.when(s < _DEG_TILES)
    def _():
        pltpu.sync_copy(stage_deg.at[pl.ds(0, _ROWS_PER_TILE)],
                        deg_sh.at[pl.ds(base, _ROWS_PER_TILE)])

    for k in range(_CHUNK // 16):
        ones_v[pl.ds(k * 16, 16)] = jnp.ones((16,), jnp.float32)

    plsc.subcore_barrier()

    def _run(y_hbm, deg_tf):
        def body(j, carry):
            cp = pltpu.async_copy(y_hbm.at[src_v.at[j]], rows_v, gsem)
            for k in range(_CHUNK // 16):
                v = dst_v[j, pl.ds(k * 16, 16)]
                idxt_v[pl.ds(k * 16, 16)] = deg_tf(v)
            cp.wait()
            pltpu.sync_copy(rows_v, acc_sh.at[dst_v.at[j]], add=True)
            pltpu.sync_copy(ones_v, deg_sh.at[idxt_v], add=True)
            return carry
        nch = jnp.where(s == 15, _LAST_CHUNKS, _FULL_CHUNKS)
        lax.fori_loop(0, nch, body, 0)

    @pl.when(c == 0)
    def _():
        _run(y0_hbm, lambda v: jnp.minimum(v, _DEG_HALF))

    @pl.when(c == 1)
    def _():
        def _tf(v):
            w = v - _DEG_HALF
            return jnp.where(w >= 0, w, _DEG_HALF)
        _run(y1_hbm, _tf)

    plsc.subcore_barrier()

    @pl.when(s < 15)
    def _():
        pltpu.sync_copy(acc_sh.at[pl.ds(base, _ROWS_PER_TILE)], stage_acc)
        pltpu.sync_copy(stage_acc, acc_out.at[c, pl.ds(base, _ROWS_PER_TILE)])

    @pl.when(s == 15)
    def _():
        pltpu.sync_copy(acc_sh.at[pl.ds(15 * _ROWS_PER_TILE, _LAST_ROWS)],
                        stage_acc.at[pl.ds(0, _LAST_ROWS)])
        pltpu.sync_copy(stage_acc.at[pl.ds(0, _LAST_ROWS)],
                        acc_out.at[c, pl.ds(15 * _ROWS_PER_TILE, _LAST_ROWS)])

    @pl.when(s < _DEG_TILES)
    def _():
        pltpu.sync_copy(deg_sh.at[pl.ds(base, _ROWS_PER_TILE)],
                        stage_deg.at[pl.ds(0, _ROWS_PER_TILE)])
        pltpu.sync_copy(stage_deg.at[pl.ds(0, _ROWS_PER_TILE)],
                        deg_out.at[pl.ds(c * _DEG_HALF + base, _ROWS_PER_TILE)])


_sc_aggregate = functools.partial(
    pl.kernel,
    out_type=(jax.ShapeDtypeStruct((_NC, _N_PAD, _DH), jnp.float32),
              jax.ShapeDtypeStruct((_N_PAD,), jnp.float32)),
    mesh=plsc.VectorSubcoreMesh(core_axis_name="c", subcore_axis_name="s"),
    compiler_params=pltpu.CompilerParams(use_tc_tiling_on_sc=False),
    scratch_types=[
        pltpu.VMEM((_CHUNKS_PER_T, _CHUNK), jnp.int32),
        pltpu.VMEM((_CHUNKS_PER_T, _CHUNK), jnp.int32),
        pltpu.VMEM((_CHUNK, _DH), jnp.float32),
        pltpu.VMEM((_CHUNK,), jnp.float32),
        pltpu.VMEM((_CHUNK,), jnp.int32),
        pltpu.VMEM((_ROWS_PER_TILE, _DH), jnp.float32),
        pltpu.VMEM((_DEG_STAGE,), jnp.float32),
        pltpu.VMEM_SHARED((_ACC_ROWS, _DH), jnp.float32),
        pltpu.VMEM_SHARED((_DEG_ROWS,), jnp.float32),
        pltpu.SemaphoreType.DMA,
    ],
)(_sc_aggregate_body)


_BLK = 128


def _neigh_mm_body(x_ref, w_ref, y0_ref, y1_ref):
    y = jnp.dot(x_ref[...], w_ref[...], preferred_element_type=jnp.float32)
    y0_ref[...] = y[:, :_DH]
    y1_ref[...] = y[:, _DH:]


def _neigh_mm(x_pad, w_neigh_t):
    return pl.pallas_call(
        _neigh_mm_body,
        grid=(_N_PAD // _BLK,),
        in_specs=[pl.BlockSpec((_BLK, _D), lambda i: (i, 0)),
                  pl.BlockSpec((_D, _D), lambda i: (0, 0))],
        out_specs=[pl.BlockSpec((_BLK, _DH), lambda i: (i, 0)),
                   pl.BlockSpec((_BLK, _DH), lambda i: (i, 0))],
        out_shape=[jax.ShapeDtypeStruct((_N_PAD, _DH), jnp.float32),
                   jax.ShapeDtypeStruct((_N_PAD, _DH), jnp.float32)],
    )(x_pad, w_neigh_t)


def _combine_body(x_ref, w_ref, b_ref, acc_ref, deg_ref, out_ref):
    deg = deg_ref[...]
    r = (1.0 / jnp.maximum(deg, 1.0)).reshape(_BLK, 1)
    h = jnp.concatenate([acc_ref[0], acc_ref[1]], axis=-1) * r
    out_ref[...] = (
        jnp.dot(x_ref[...], w_ref[...], preferred_element_type=jnp.float32)
        + h + b_ref[...]
    )


def _combine(x_pad, w_self_t, b2d, acc, deg2d):
    return pl.pallas_call(
        _combine_body,
        grid=(_N_PAD // _BLK,),
        in_specs=[pl.BlockSpec((_BLK, _D), lambda i: (i, 0)),
                  pl.BlockSpec((_D, _D), lambda i: (0, 0)),
                  pl.BlockSpec((1, _D), lambda i: (0, 0)),
                  pl.BlockSpec((_NC, _BLK, _DH), lambda i: (0, i, 0)),
                  pl.BlockSpec((1, _BLK), lambda i: (0, i))],
        out_specs=pl.BlockSpec((_BLK, _D), lambda i: (i, 0)),
        out_shape=jax.ShapeDtypeStruct((_N_PAD, _D), jnp.float32),
    )(x_pad, w_self_t, b2d, acc, deg2d)


def kernel(x, edge_index, W_self, W_neigh, b):
    x_pad = jnp.concatenate(
        [x, jnp.zeros((_N_PAD - _N, _D), jnp.float32)], axis=0)
    src = edge_index[0]
    dst = edge_index[1]
    pad_e = _E_PAD - _E
    src_p = jnp.concatenate(
        [src, jnp.zeros((pad_e,), jnp.int32)]).reshape(_NS, _CHUNKS_PER_T, _CHUNK)
    dst_p = jnp.concatenate(
        [dst, jnp.full((pad_e,), _N, jnp.int32)]).reshape(_NS, _CHUNKS_PER_T, _CHUNK)

    y0, y1 = _neigh_mm(x_pad, W_neigh.T)

    acc, deg = _sc_aggregate(y0, y1, src_p, dst_p)

    out = _combine(x_pad, W_self.T, b.reshape(1, _D), acc, deg.reshape(1, _N_PAD))
    return out[:_N]


# speedup vs baseline: 5.9941x; 1.0333x over previous
import functools

import jax
import jax.numpy as jnp
from jax import lax
from jax.experimental import pallas as pl
from jax.experimental.pallas import tpu as pltpu
from jax.experimental.pallas import tpu_sc as plsc

_N = 10000
_E = 320000
_D = 128
_DH = _D // 2

_NC = 2
_NS = 16
_NW = _NC * _NS

_CHUNK = 64
_CHUNKS_PER_T = 320
_EDGES_PER_T = _CHUNK * _CHUNKS_PER_T
_E_PAD = _EDGES_PER_T * _NS
_N_PAD = 10112
_ROWS_PER_TILE = _N_PAD // _NS
_ACC_ROWS = _N
_LAST_ROWS = _ACC_ROWS - 15 * _ROWS_PER_TILE
_FULL_CHUNKS = _CHUNKS_PER_T
_LAST_CHUNKS = (_E - 15 * _EDGES_PER_T) // _CHUNK
_DEG_STAGE = 640
_DEG_HALF = _N_PAD // 2
_DEG_ROWS = _DEG_HALF + 64
_DEG_TILES = _DEG_HALF // _ROWS_PER_TILE


def _sc_aggregate_body(y0_hbm, y1_hbm, src_hbm, dst_hbm,
                       acc_out, deg_out,
                       src_v, dst_v, rows_v0, rows_v1, ones_v, idxa_v, idxt_v,
                       stage_acc, stage_deg, acc_sh, deg_sh, gsem0, gsem1):
    c = lax.axis_index("c")
    s = lax.axis_index("s")
    base = s * _ROWS_PER_TILE

    pltpu.sync_copy(src_hbm.at[s], src_v)
    pltpu.sync_copy(dst_hbm.at[s], dst_v)

    z16 = jnp.zeros((16,), jnp.float32)

    def _zrow(i, carry):
        for k in range(_DH // 16):
            stage_acc[i, pl.ds(k * 16, 16)] = z16
        return carry

    lax.fori_loop(0, _ROWS_PER_TILE, _zrow, 0)

    def _zdeg(i, carry):
        stage_deg[pl.ds(i * 16, 16)] = z16
        return carry

    lax.fori_loop(0, _DEG_STAGE // 16, _zdeg, 0)

    @pl.when(s < 15)
    def _():
        pltpu.sync_copy(stage_acc, acc_sh.at[pl.ds(base, _ROWS_PER_TILE)])

    @pl.when(s == 15)
    def _():
        pltpu.sync_copy(stage_acc.at[pl.ds(0, _LAST_ROWS)],
                        acc_sh.at[pl.ds(15 * _ROWS_PER_TILE, _LAST_ROWS)])

    @pl.when(s < _DEG_TILES)
    def _():
        pltpu.sync_copy(stage_deg.at[pl.ds(0, _ROWS_PER_TILE)],
                        deg_sh.at[pl.ds(base, _ROWS_PER_TILE)])

    for k in range(_CHUNK // 16):
        ones_v[pl.ds(k * 16, 16)] = jnp.ones((16,), jnp.float32)

    plsc.subcore_barrier()

    def _run(y_hbm, deg_tf):
        rows = (rows_v0, rows_v1)
        sems = (gsem0, gsem1)

        def _start(j, b):
            pltpu.async_copy(y_hbm.at[src_v.at[j]], rows[b], sems[b])

        def _drain(b):
            pltpu.make_async_copy(y_hbm.at[src_v.at[0]], rows[b], sems[b]).wait()

        def _scatter(j, b):
            for k in range(_CHUNK // 16):
                v = dst_v[j, pl.ds(k * 16, 16)]
                idxa_v[pl.ds(k * 16, 16)] = v
                idxt_v[pl.ds(k * 16, 16)] = deg_tf(v)
            _drain(b)
            pltpu.sync_copy(rows[b], acc_sh.at[idxa_v], add=True)
            pltpu.sync_copy(ones_v, deg_sh.at[idxt_v], add=True)

        nch = jnp.where(s == 15, _LAST_CHUNKS, _FULL_CHUNKS)
        npairs = nch // 2
        _start(0, 0)

        def pair(p, carry):
            j0 = 2 * p
            _start(j0 + 1, 1)
            _scatter(j0, 0)

            @pl.when(p + 1 < npairs)
            def _():
                _start(j0 + 2, 0)

            _scatter(j0 + 1, 1)
            return carry

        lax.fori_loop(0, npairs, pair, 0)

    @pl.when(c == 0)
    def _():
        _run(y0_hbm, lambda v: jnp.minimum(v, _DEG_HALF))

    @pl.when(c == 1)
    def _():
        def _tf(v):
            w = v - _DEG_HALF
            return jnp.where(w >= 0, w, _DEG_HALF)
        _run(y1_hbm, _tf)

    plsc.subcore_barrier()

    @pl.when(s < 15)
    def _():
        pltpu.sync_copy(acc_sh.at[pl.ds(base, _ROWS_PER_TILE)], stage_acc)
        pltpu.sync_copy(stage_acc, acc_out.at[c, pl.ds(base, _ROWS_PER_TILE)])

    @pl.when(s == 15)
    def _():
        pltpu.sync_copy(acc_sh.at[pl.ds(15 * _ROWS_PER_TILE, _LAST_ROWS)],
                        stage_acc.at[pl.ds(0, _LAST_ROWS)])
        pltpu.sync_copy(stage_acc.at[pl.ds(0, _LAST_ROWS)],
                        acc_out.at[c, pl.ds(15 * _ROWS_PER_TILE, _LAST_ROWS)])

    @pl.when(s < _DEG_TILES)
    def _():
        pltpu.sync_copy(deg_sh.at[pl.ds(base, _ROWS_PER_TILE)],
                        stage_deg.at[pl.ds(0, _ROWS_PER_TILE)])
        pltpu.sync_copy(stage_deg.at[pl.ds(0, _ROWS_PER_TILE)],
                        deg_out.at[pl.ds(c * _DEG_HALF + base, _ROWS_PER_TILE)])


_sc_aggregate = functools.partial(
    pl.kernel,
    out_type=(jax.ShapeDtypeStruct((_NC, _N_PAD, _DH), jnp.float32),
              jax.ShapeDtypeStruct((_N_PAD,), jnp.float32)),
    mesh=plsc.VectorSubcoreMesh(core_axis_name="c", subcore_axis_name="s"),
    compiler_params=pltpu.CompilerParams(use_tc_tiling_on_sc=False),
    scratch_types=[
        pltpu.VMEM((_CHUNKS_PER_T, _CHUNK), jnp.int32),
        pltpu.VMEM((_CHUNKS_PER_T, _CHUNK), jnp.int32),
        pltpu.VMEM((_CHUNK, _DH), jnp.float32),
        pltpu.VMEM((_CHUNK, _DH), jnp.float32),
        pltpu.VMEM((_CHUNK,), jnp.float32),
        pltpu.VMEM((_CHUNK,), jnp.int32),
        pltpu.VMEM((_CHUNK,), jnp.int32),
        pltpu.VMEM((_ROWS_PER_TILE, _DH), jnp.float32),
        pltpu.VMEM((_DEG_STAGE,), jnp.float32),
        pltpu.VMEM_SHARED((_ACC_ROWS, _DH), jnp.float32),
        pltpu.VMEM_SHARED((_DEG_ROWS,), jnp.float32),
        pltpu.SemaphoreType.DMA,
        pltpu.SemaphoreType.DMA,
    ],
)(_sc_aggregate_body)


_BLK = 128


def _neigh_mm_body(x_ref, w_ref, y0_ref, y1_ref):
    y = jnp.dot(x_ref[...], w_ref[...], preferred_element_type=jnp.float32)
    y0_ref[...] = y[:, :_DH]
    y1_ref[...] = y[:, _DH:]


def _neigh_mm(x_pad, w_neigh_t):
    return pl.pallas_call(
        _neigh_mm_body,
        grid=(_N_PAD // _BLK,),
        in_specs=[pl.BlockSpec((_BLK, _D), lambda i: (i, 0)),
                  pl.BlockSpec((_D, _D), lambda i: (0, 0))],
        out_specs=[pl.BlockSpec((_BLK, _DH), lambda i: (i, 0)),
                   pl.BlockSpec((_BLK, _DH), lambda i: (i, 0))],
        out_shape=[jax.ShapeDtypeStruct((_N_PAD, _DH), jnp.float32),
                   jax.ShapeDtypeStruct((_N_PAD, _DH), jnp.float32)],
    )(x_pad, w_neigh_t)


def _combine_body(x_ref, w_ref, b_ref, acc_ref, deg_ref, out_ref):
    deg = deg_ref[...]
    r = (1.0 / jnp.maximum(deg, 1.0)).reshape(_BLK, 1)
    h = jnp.concatenate([acc_ref[0], acc_ref[1]], axis=-1) * r
    out_ref[...] = (
        jnp.dot(x_ref[...], w_ref[...], preferred_element_type=jnp.float32)
        + h + b_ref[...]
    )


def _combine(x_pad, w_self_t, b2d, acc, deg2d):
    return pl.pallas_call(
        _combine_body,
        grid=(_N_PAD // _BLK,),
        in_specs=[pl.BlockSpec((_BLK, _D), lambda i: (i, 0)),
                  pl.BlockSpec((_D, _D), lambda i: (0, 0)),
                  pl.BlockSpec((1, _D), lambda i: (0, 0)),
                  pl.BlockSpec((_NC, _BLK, _DH), lambda i: (0, i, 0)),
                  pl.BlockSpec((1, _BLK), lambda i: (0, i))],
        out_specs=pl.BlockSpec((_BLK, _D), lambda i: (i, 0)),
        out_shape=jax.ShapeDtypeStruct((_N_PAD, _D), jnp.float32),
    )(x_pad, w_self_t, b2d, acc, deg2d)


def kernel(x, edge_index, W_self, W_neigh, b):
    x_pad = jnp.concatenate(
        [x, jnp.zeros((_N_PAD - _N, _D), jnp.float32)], axis=0)
    src = edge_index[0]
    dst = edge_index[1]
    pad_e = _E_PAD - _E
    src_p = jnp.concatenate(
        [src, jnp.zeros((pad_e,), jnp.int32)]).reshape(_NS, _CHUNKS_PER_T, _CHUNK)
    dst_p = jnp.concatenate(
        [dst, jnp.full((pad_e,), _N, jnp.int32)]).reshape(_NS, _CHUNKS_PER_T, _CHUNK)

    y0, y1 = _neigh_mm(x_pad, W_neigh.T)

    acc, deg = _sc_aggregate(y0, y1, src_p, dst_p)

    out = _combine(x_pad, W_self.T, b.reshape(1, _D), acc, deg.reshape(1, _N_PAD))
    return out[:_N]


# speedup vs baseline: 6.0327x; 1.0064x over previous
import functools

import jax
import jax.numpy as jnp
from jax import lax
from jax.experimental import pallas as pl
from jax.experimental.pallas import tpu as pltpu
from jax.experimental.pallas import tpu_sc as plsc

_N = 10000
_E = 320000
_D = 128
_DH = _D // 2

_NC = 2
_NS = 16
_NW = _NC * _NS

_CHUNK = 128
_CHUNKS_PER_T = 160
_WIN = 20
_EDGES_PER_T = _CHUNK * _CHUNKS_PER_T
_E_PAD = _EDGES_PER_T * _NS
_N_PAD = 10112
_ROWS_PER_TILE = _N_PAD // _NS
_ACC_ROWS = _N
_LAST_ROWS = _ACC_ROWS - 15 * _ROWS_PER_TILE
_FULL_CHUNKS = _CHUNKS_PER_T
_LAST_CHUNKS = (_E - 15 * _EDGES_PER_T) // _CHUNK
_DEG_STAGE = 640
_DEG_HALF = _N_PAD // 2
_DEG_ROWS = _DEG_HALF + 64
_DEG_TILES = _DEG_HALF // _ROWS_PER_TILE


def _sc_aggregate_body(y0_hbm, y1_hbm, src_hbm, dst_hbm,
                       acc_out, deg_out,
                       srcw_v, dstw_v, rows_v0, rows_v1, ones_v, idxa_v, idxt_v,
                       stage_acc, stage_deg, acc_sh, deg_sh, gsem0, gsem1, wsem):
    c = lax.axis_index("c")
    s = lax.axis_index("s")
    base = s * _ROWS_PER_TILE

    z16 = jnp.zeros((16,), jnp.float32)

    def _zrow(i, carry):
        for k in range(_DH // 16):
            stage_acc[i, pl.ds(k * 16, 16)] = z16
        return carry

    lax.fori_loop(0, _ROWS_PER_TILE, _zrow, 0)

    def _zdeg(i, carry):
        stage_deg[pl.ds(i * 16, 16)] = z16
        return carry

    lax.fori_loop(0, _DEG_STAGE // 16, _zdeg, 0)

    @pl.when(s < 15)
    def _():
        pltpu.sync_copy(stage_acc, acc_sh.at[pl.ds(base, _ROWS_PER_TILE)])

    @pl.when(s == 15)
    def _():
        pltpu.sync_copy(stage_acc.at[pl.ds(0, _LAST_ROWS)],
                        acc_sh.at[pl.ds(15 * _ROWS_PER_TILE, _LAST_ROWS)])

    @pl.when(s < _DEG_TILES)
    def _():
        pltpu.sync_copy(stage_deg.at[pl.ds(0, _ROWS_PER_TILE)],
                        deg_sh.at[pl.ds(base, _ROWS_PER_TILE)])

    for k in range(_CHUNK // 16):
        ones_v[pl.ds(k * 16, 16)] = jnp.ones((16,), jnp.float32)

    plsc.subcore_barrier()

    def _run(y_hbm, deg_tf):
        rows = (rows_v0, rows_v1)
        sems = (gsem0, gsem1)

        def _stage(w, q):
            pltpu.async_copy(src_hbm.at[s, pl.ds(w * _WIN, _WIN)],
                             srcw_v.at[pl.ds(q * _WIN, _WIN)], wsem)
            pltpu.async_copy(dst_hbm.at[s, pl.ds(w * _WIN, _WIN)],
                             dstw_v.at[pl.ds(q * _WIN, _WIN)], wsem)

        def _drain_stage():
            pltpu.make_async_copy(src_hbm.at[s, pl.ds(0, _WIN)],
                                  srcw_v.at[pl.ds(0, _WIN)], wsem).wait()
            pltpu.make_async_copy(dst_hbm.at[s, pl.ds(0, _WIN)],
                                  dstw_v.at[pl.ds(0, _WIN)], wsem).wait()

        def _start(r, b):
            pltpu.async_copy(y_hbm.at[srcw_v.at[r]], rows[b], sems[b])

        def _drain(b):
            pltpu.make_async_copy(y_hbm.at[srcw_v.at[0]], rows[b],
                                  sems[b]).wait()

        def _scatter(r, b):
            for k in range(_CHUNK // 16):
                v = dstw_v[r, pl.ds(k * 16, 16)]
                idxa_v[pl.ds(k * 16, 16)] = v
                idxt_v[pl.ds(k * 16, 16)] = deg_tf(v)
            _drain(b)
            pltpu.sync_copy(rows[b], acc_sh.at[idxa_v], add=True)
            pltpu.sync_copy(ones_v, deg_sh.at[idxt_v], add=True)

        nwin = jnp.where(s == 15, _LAST_CHUNKS // _WIN, _FULL_CHUNKS // _WIN)
        _stage(0, 0)
        _drain_stage()

        def win(w, carry):
            q = lax.rem(w, 2)
            rbase = q * _WIN

            @pl.when(w + 1 < nwin)
            def _():
                _stage(w + 1, 1 - q)

            _start(rbase, 0)

            def pair(p, carry2):
                r0 = rbase + 2 * p
                _start(r0 + 1, 1)
                _scatter(r0, 0)

                @pl.when(p + 1 < _WIN // 2)
                def _():
                    _start(r0 + 2, 0)

                _scatter(r0 + 1, 1)
                return carry2

            lax.fori_loop(0, _WIN // 2, pair, 0)

            @pl.when(w + 1 < nwin)
            def _():
                _drain_stage()

            return carry

        lax.fori_loop(0, nwin, win, 0)

    @pl.when(c == 0)
    def _():
        _run(y0_hbm, lambda v: jnp.minimum(v, _DEG_HALF))

    @pl.when(c == 1)
    def _():
        def _tf(v):
            w = v - _DEG_HALF
            return jnp.where(w >= 0, w, _DEG_HALF)
        _run(y1_hbm, _tf)

    plsc.subcore_barrier()

    @pl.when(s < 15)
    def _():
        pltpu.sync_copy(acc_sh.at[pl.ds(base, _ROWS_PER_TILE)], stage_acc)
        pltpu.sync_copy(stage_acc, acc_out.at[c, pl.ds(base, _ROWS_PER_TILE)])

    @pl.when(s == 15)
    def _():
        pltpu.sync_copy(acc_sh.at[pl.ds(15 * _ROWS_PER_TILE, _LAST_ROWS)],
                        stage_acc.at[pl.ds(0, _LAST_ROWS)])
        pltpu.sync_copy(stage_acc.at[pl.ds(0, _LAST_ROWS)],
                        acc_out.at[c, pl.ds(15 * _ROWS_PER_TILE, _LAST_ROWS)])

    @pl.when(s < _DEG_TILES)
    def _():
        pltpu.sync_copy(deg_sh.at[pl.ds(base, _ROWS_PER_TILE)],
                        stage_deg.at[pl.ds(0, _ROWS_PER_TILE)])
        pltpu.sync_copy(stage_deg.at[pl.ds(0, _ROWS_PER_TILE)],
                        deg_out.at[pl.ds(c * _DEG_HALF + base, _ROWS_PER_TILE)])


_sc_aggregate = functools.partial(
    pl.kernel,
    out_type=(jax.ShapeDtypeStruct((_NC, _N_PAD, _DH), jnp.float32),
              jax.ShapeDtypeStruct((_N_PAD,), jnp.float32)),
    mesh=plsc.VectorSubcoreMesh(core_axis_name="c", subcore_axis_name="s"),
    compiler_params=pltpu.CompilerParams(use_tc_tiling_on_sc=False),
    scratch_types=[
        pltpu.VMEM((2 * _WIN, _CHUNK), jnp.int32),
        pltpu.VMEM((2 * _WIN, _CHUNK), jnp.int32),
        pltpu.VMEM((_CHUNK, _DH), jnp.float32),
        pltpu.VMEM((_CHUNK, _DH), jnp.float32),
        pltpu.VMEM((_CHUNK,), jnp.float32),
        pltpu.VMEM((_CHUNK,), jnp.int32),
        pltpu.VMEM((_CHUNK,), jnp.int32),
        pltpu.VMEM((_ROWS_PER_TILE, _DH), jnp.float32),
        pltpu.VMEM((_DEG_STAGE,), jnp.float32),
        pltpu.VMEM_SHARED((_ACC_ROWS, _DH), jnp.float32),
        pltpu.VMEM_SHARED((_DEG_ROWS,), jnp.float32),
        pltpu.SemaphoreType.DMA,
        pltpu.SemaphoreType.DMA,
        pltpu.SemaphoreType.DMA,
    ],
)(_sc_aggregate_body)


_BLK = 128


def _neigh_mm_body(x_ref, w_ref, y0_ref, y1_ref):
    y = jnp.dot(x_ref[...], w_ref[...], preferred_element_type=jnp.float32)
    y0_ref[...] = y[:, :_DH]
    y1_ref[...] = y[:, _DH:]


def _neigh_mm(x_pad, w_neigh_t):
    return pl.pallas_call(
        _neigh_mm_body,
        grid=(_N_PAD // _BLK,),
        in_specs=[pl.BlockSpec((_BLK, _D), lambda i: (i, 0)),
                  pl.BlockSpec((_D, _D), lambda i: (0, 0))],
        out_specs=[pl.BlockSpec((_BLK, _DH), lambda i: (i, 0)),
                   pl.BlockSpec((_BLK, _DH), lambda i: (i, 0))],
        out_shape=[jax.ShapeDtypeStruct((_N_PAD, _DH), jnp.float32),
                   jax.ShapeDtypeStruct((_N_PAD, _DH), jnp.float32)],
    )(x_pad, w_neigh_t)


def _combine_body(x_ref, w_ref, b_ref, acc_ref, deg_ref, out_ref):
    deg = deg_ref[...]
    r = (1.0 / jnp.maximum(deg, 1.0)).reshape(_BLK, 1)
    h = jnp.concatenate([acc_ref[0], acc_ref[1]], axis=-1) * r
    out_ref[...] = (
        jnp.dot(x_ref[...], w_ref[...], preferred_element_type=jnp.float32)
        + h + b_ref[...]
    )


def _combine(x_pad, w_self_t, b2d, acc, deg2d):
    return pl.pallas_call(
        _combine_body,
        grid=(_N_PAD // _BLK,),
        in_specs=[pl.BlockSpec((_BLK, _D), lambda i: (i, 0)),
                  pl.BlockSpec((_D, _D), lambda i: (0, 0)),
                  pl.BlockSpec((1, _D), lambda i: (0, 0)),
                  pl.BlockSpec((_NC, _BLK, _DH), lambda i: (0, i, 0)),
                  pl.BlockSpec((1, _BLK), lambda i: (0, i))],
        out_specs=pl.BlockSpec((_BLK, _D), lambda i: (i, 0)),
        out_shape=jax.ShapeDtypeStruct((_N_PAD, _D), jnp.float32),
    )(x_pad, w_self_t, b2d, acc, deg2d)


def kernel(x, edge_index, W_self, W_neigh, b):
    x_pad = jnp.concatenate(
        [x, jnp.zeros((_N_PAD - _N, _D), jnp.float32)], axis=0)
    src = edge_index[0]
    dst = edge_index[1]
    pad_e = _E_PAD - _E
    src_p = jnp.concatenate(
        [src, jnp.zeros((pad_e,), jnp.int32)]).reshape(_NS, _CHUNKS_PER_T, _CHUNK)
    dst_p = jnp.concatenate(
        [dst, jnp.full((pad_e,), _N, jnp.int32)]).reshape(_NS, _CHUNKS_PER_T, _CHUNK)

    y0, y1 = _neigh_mm(x_pad, W_neigh.T)

    acc, deg = _sc_aggregate(y0, y1, src_p, dst_p)

    out = _combine(x_pad, W_self.T, b.reshape(1, _D), acc, deg.reshape(1, _N_PAD))
    return out[:_N]


# speedup vs baseline: 7.3012x; 1.2103x over previous
import functools

import jax
import jax.numpy as jnp
from jax import lax
from jax.experimental import pallas as pl
from jax.experimental.pallas import tpu as pltpu
from jax.experimental.pallas import tpu_sc as plsc

_N = 10000
_E = 320000
_D = 128
_DH = _D // 2
_DA = _DH + 16

_NC = 2
_NS = 16
_NW = _NC * _NS

_CHUNK = 128
_CHUNKS_PER_T = 160
_WIN = 10
_EDGES_PER_T = _CHUNK * _CHUNKS_PER_T
_E_PAD = _EDGES_PER_T * _NS
_N_PAD = 10112
_ROWS_PER_TILE = _N_PAD // _NS
_ACC_ROWS = _N
_LAST_ROWS = _ACC_ROWS - 15 * _ROWS_PER_TILE
_FULL_CHUNKS = _CHUNKS_PER_T
_LAST_CHUNKS = (_E - 15 * _EDGES_PER_T) // _CHUNK


def _sc_aggregate_body(y0_hbm, y1_hbm, src_hbm, dst_hbm, acc_out,
                       srcw_v, dstw_v, rows_v0, rows_v1,
                       stage_acc, acc_sh, gsem0, gsem1, wsem):
    c = lax.axis_index("c")
    s = lax.axis_index("s")
    base = s * _ROWS_PER_TILE

    z16 = jnp.zeros((16,), jnp.float32)

    def _zrow(i, carry):
        for k in range(_DA // 16):
            stage_acc[i, pl.ds(k * 16, 16)] = z16
        return carry

    lax.fori_loop(0, _ROWS_PER_TILE, _zrow, 0)

    @pl.when(s < 15)
    def _():
        pltpu.sync_copy(stage_acc, acc_sh.at[pl.ds(base, _ROWS_PER_TILE)])

    @pl.when(s == 15)
    def _():
        pltpu.sync_copy(stage_acc.at[pl.ds(0, _LAST_ROWS)],
                        acc_sh.at[pl.ds(15 * _ROWS_PER_TILE, _LAST_ROWS)])

    plsc.subcore_barrier()

    def _run(y_hbm):
        rows = (rows_v0, rows_v1)
        sems = (gsem0, gsem1)

        def _stage(w, q):
            pltpu.async_copy(src_hbm.at[s, pl.ds(w * _WIN, _WIN)],
                             srcw_v.at[pl.ds(q * _WIN, _WIN)], wsem)
            pltpu.async_copy(dst_hbm.at[s, pl.ds(w * _WIN, _WIN)],
                             dstw_v.at[pl.ds(q * _WIN, _WIN)], wsem)

        def _drain_stage():
            pltpu.make_async_copy(src_hbm.at[s, pl.ds(0, _WIN)],
                                  srcw_v.at[pl.ds(0, _WIN)], wsem).wait()
            pltpu.make_async_copy(dst_hbm.at[s, pl.ds(0, _WIN)],
                                  dstw_v.at[pl.ds(0, _WIN)], wsem).wait()

        def _start(r, b):
            pltpu.async_copy(y_hbm.at[srcw_v.at[r]], rows[b], sems[b])

        def _scatter(r, b):
            pltpu.make_async_copy(y_hbm.at[srcw_v.at[0]], rows[b],
                                  sems[b]).wait()
            pltpu.sync_copy(rows[b], acc_sh.at[dstw_v.at[r]], add=True)

        nwin = jnp.where(s == 15, _LAST_CHUNKS // _WIN, _FULL_CHUNKS // _WIN)
        _stage(0, 0)
        _drain_stage()

        def win(w, carry):
            q = lax.rem(w, 2)
            rbase = q * _WIN

            @pl.when(w + 1 < nwin)
            def _():
                _stage(w + 1, 1 - q)

            _start(rbase, 0)

            def pair(p, carry2):
                r0 = rbase + 2 * p
                _start(r0 + 1, 1)
                _scatter(r0, 0)

                @pl.when(p + 1 < _WIN // 2)
                def _():
                    _start(r0 + 2, 0)

                _scatter(r0 + 1, 1)
                return carry2

            lax.fori_loop(0, _WIN // 2, pair, 0)

            @pl.when(w + 1 < nwin)
            def _():
                _drain_stage()

            return carry

        lax.fori_loop(0, nwin, win, 0)

    @pl.when(c == 0)
    def _():
        _run(y0_hbm)

    @pl.when(c == 1)
    def _():
        _run(y1_hbm)

    plsc.subcore_barrier()

    @pl.when(s < 15)
    def _():
        pltpu.sync_copy(acc_sh.at[pl.ds(base, _ROWS_PER_TILE)], stage_acc)
        pltpu.sync_copy(stage_acc, acc_out.at[c, pl.ds(base, _ROWS_PER_TILE)])

    @pl.when(s == 15)
    def _():
        pltpu.sync_copy(acc_sh.at[pl.ds(15 * _ROWS_PER_TILE, _LAST_ROWS)],
                        stage_acc.at[pl.ds(0, _LAST_ROWS)])
        pltpu.sync_copy(stage_acc.at[pl.ds(0, _LAST_ROWS)],
                        acc_out.at[c, pl.ds(15 * _ROWS_PER_TILE, _LAST_ROWS)])


_sc_aggregate = functools.partial(
    pl.kernel,
    out_type=jax.ShapeDtypeStruct((_NC, _N_PAD, _DA), jnp.float32),
    mesh=plsc.VectorSubcoreMesh(core_axis_name="c", subcore_axis_name="s"),
    compiler_params=pltpu.CompilerParams(use_tc_tiling_on_sc=False),
    scratch_types=[
        pltpu.VMEM((2 * _WIN, _CHUNK), jnp.int32),
        pltpu.VMEM((2 * _WIN, _CHUNK), jnp.int32),
        pltpu.VMEM((_CHUNK, _DA), jnp.float32),
        pltpu.VMEM((_CHUNK, _DA), jnp.float32),
        pltpu.VMEM((_ROWS_PER_TILE, _DA), jnp.float32),
        pltpu.VMEM_SHARED((_ACC_ROWS, _DA), jnp.float32),
        pltpu.SemaphoreType.DMA,
        pltpu.SemaphoreType.DMA,
        pltpu.SemaphoreType.DMA,
    ],
)(_sc_aggregate_body)


_BLK = 128


def _neigh_mm_body(x_ref, w_ref, y0_ref, y1_ref):
    y = jnp.dot(x_ref[...], w_ref[...], preferred_element_type=jnp.float32)
    ones = jnp.ones((_BLK, _DA - _DH), jnp.float32)
    y0_ref[...] = jnp.concatenate([y[:, :_DH], ones], axis=1)
    y1_ref[...] = jnp.concatenate([y[:, _DH:], ones], axis=1)


def _neigh_mm(x_pad, w_neigh_t):
    return pl.pallas_call(
        _neigh_mm_body,
        grid=(_N_PAD // _BLK,),
        in_specs=[pl.BlockSpec((_BLK, _D), lambda i: (i, 0)),
                  pl.BlockSpec((_D, _D), lambda i: (0, 0))],
        out_specs=[pl.BlockSpec((_BLK, _DA), lambda i: (i, 0)),
                   pl.BlockSpec((_BLK, _DA), lambda i: (i, 0))],
        out_shape=[jax.ShapeDtypeStruct((_N_PAD, _DA), jnp.float32),
                   jax.ShapeDtypeStruct((_N_PAD, _DA), jnp.float32)],
    )(x_pad, w_neigh_t)


def _combine_body(x_ref, w_ref, b_ref, acc_ref, out_ref):
    a0 = acc_ref[0]
    a1 = acc_ref[1]
    deg = a0[:, _DH:_DH + 1]
    r = 1.0 / jnp.maximum(deg, 1.0)
    h = jnp.concatenate([a0[:, :_DH], a1[:, :_DH]], axis=1) * r
    out_ref[...] = (
        jnp.dot(x_ref[...], w_ref[...], preferred_element_type=jnp.float32)
        + h + b_ref[...]
    )


def _combine(x_pad, w_self_t, b2d, acc):
    return pl.pallas_call(
        _combine_body,
        grid=(_N_PAD // _BLK,),
        in_specs=[pl.BlockSpec((_BLK, _D), lambda i: (i, 0)),
                  pl.BlockSpec((_D, _D), lambda i: (0, 0)),
                  pl.BlockSpec((1, _D), lambda i: (0, 0)),
                  pl.BlockSpec((_NC, _BLK, _DA), lambda i: (0, i, 0))],
        out_specs=pl.BlockSpec((_BLK, _D), lambda i: (i, 0)),
        out_shape=jax.ShapeDtypeStruct((_N_PAD, _D), jnp.float32),
    )(x_pad, w_self_t, b2d, acc)


def kernel(x, edge_index, W_self, W_neigh, b):
    x_pad = jnp.concatenate(
        [x, jnp.zeros((_N_PAD - _N, _D), jnp.float32)], axis=0)
    src = edge_index[0]
    dst = edge_index[1]
    pad_e = _E_PAD - _E
    src_p = jnp.concatenate(
        [src, jnp.zeros((pad_e,), jnp.int32)]).reshape(_NS, _CHUNKS_PER_T, _CHUNK)
    dst_p = jnp.concatenate(
        [dst, jnp.full((pad_e,), _N, jnp.int32)]).reshape(_NS, _CHUNKS_PER_T, _CHUNK)

    y0, y1 = _neigh_mm(x_pad, W_neigh.T)

    acc = _sc_aggregate(y0, y1, src_p, dst_p)

    out = _combine(x_pad, W_self.T, b.reshape(1, _D), acc)
    return out[:_N]


# speedup vs baseline: 7.7635x; 1.0633x over previous
import functools

import jax
import jax.numpy as jnp
from jax import lax
from jax.experimental import pallas as pl
from jax.experimental.pallas import tpu as pltpu
from jax.experimental.pallas import tpu_sc as plsc

_N = 10000
_E = 320000
_D = 128
_DH = _D // 2
_DA = _DH + 16

_NC = 2
_NS = 16

_CHUNK = 128
_CHUNKS_PER_T = 160
_WIN = 10
_ECHUNKS = _E // _CHUNK
_N_PAD = 10112
_ROWS_PER_TILE = _N_PAD // _NS
_ACC_ROWS = _N
_LAST_ROWS = _ACC_ROWS - 15 * _ROWS_PER_TILE
_FULL_CHUNKS = _CHUNKS_PER_T
_LAST_CHUNKS = _ECHUNKS - 15 * _CHUNKS_PER_T


def _sc_aggregate_body(y0_hbm, y1_hbm, ei_hbm, acc_out,
                       srcw_v, dstw_v, rows_v0, rows_v1,
                       stage_acc, acc_sh, gsem0, gsem1, wsem):
    c = lax.axis_index("c")
    s = lax.axis_index("s")
    base = s * _ROWS_PER_TILE
    cbase = s * _CHUNKS_PER_T

    z16 = jnp.zeros((16,), jnp.float32)

    def _zrow(i, carry):
        for k in range(_DA // 16):
            stage_acc[i, pl.ds(k * 16, 16)] = z16
        return carry

    lax.fori_loop(0, _ROWS_PER_TILE, _zrow, 0)

    @pl.when(s < 15)
    def _():
        pltpu.sync_copy(stage_acc, acc_sh.at[pl.ds(base, _ROWS_PER_TILE)])

    @pl.when(s == 15)
    def _():
        pltpu.sync_copy(stage_acc.at[pl.ds(0, _LAST_ROWS)],
                        acc_sh.at[pl.ds(15 * _ROWS_PER_TILE, _LAST_ROWS)])

    plsc.subcore_barrier()

    def _run(y_hbm):
        rows = (rows_v0, rows_v1)
        sems = (gsem0, gsem1)

        def _stage(w, q):
            pltpu.async_copy(ei_hbm.at[0, pl.ds(cbase + w * _WIN, _WIN)],
                             srcw_v.at[pl.ds(q * _WIN, _WIN)], wsem)
            pltpu.async_copy(ei_hbm.at[1, pl.ds(cbase + w * _WIN, _WIN)],
                             dstw_v.at[pl.ds(q * _WIN, _WIN)], wsem)

        def _drain_stage():
            pltpu.make_async_copy(ei_hbm.at[0, pl.ds(0, _WIN)],
                                  srcw_v.at[pl.ds(0, _WIN)], wsem).wait()
            pltpu.make_async_copy(ei_hbm.at[1, pl.ds(0, _WIN)],
                                  dstw_v.at[pl.ds(0, _WIN)], wsem).wait()

        def _start(r, b):
            pltpu.async_copy(y_hbm.at[srcw_v.at[r]], rows[b], sems[b])

        def _scatter(r, b):
            pltpu.make_async_copy(y_hbm.at[srcw_v.at[0]], rows[b],
                                  sems[b]).wait()
            pltpu.sync_copy(rows[b], acc_sh.at[dstw_v.at[r]], add=True)

        nwin = jnp.where(s == 15, _LAST_CHUNKS // _WIN, _FULL_CHUNKS // _WIN)
        _stage(0, 0)
        _drain_stage()

        def win(w, carry):
            q = lax.rem(w, 2)
            rbase = q * _WIN

            @pl.when(w + 1 < nwin)
            def _():
                _stage(w + 1, 1 - q)

            _start(rbase, 0)

            def pair(p, carry2):
                r0 = rbase + 2 * p
                _start(r0 + 1, 1)
                _scatter(r0, 0)

                @pl.when(p + 1 < _WIN // 2)
                def _():
                    _start(r0 + 2, 0)

                _scatter(r0 + 1, 1)
                return carry2

            lax.fori_loop(0, _WIN // 2, pair, 0)

            @pl.when(w + 1 < nwin)
            def _():
                _drain_stage()

            return carry

        lax.fori_loop(0, nwin, win, 0)

    @pl.when(c == 0)
    def _():
        _run(y0_hbm)

    @pl.when(c == 1)
    def _():
        _run(y1_hbm)

    plsc.subcore_barrier()

    @pl.when(s < 15)
    def _():
        pltpu.sync_copy(acc_sh.at[pl.ds(base, _ROWS_PER_TILE)], stage_acc)
        pltpu.sync_copy(stage_acc, acc_out.at[c, pl.ds(base, _ROWS_PER_TILE)])

    @pl.when(s == 15)
    def _():
        pltpu.sync_copy(acc_sh.at[pl.ds(15 * _ROWS_PER_TILE, _LAST_ROWS)],
                        stage_acc.at[pl.ds(0, _LAST_ROWS)])
        pltpu.sync_copy(stage_acc.at[pl.ds(0, _LAST_ROWS)],
                        acc_out.at[c, pl.ds(15 * _ROWS_PER_TILE, _LAST_ROWS)])


_sc_aggregate = functools.partial(
    pl.kernel,
    out_type=jax.ShapeDtypeStruct((_NC, _ACC_ROWS, _DA), jnp.float32),
    mesh=plsc.VectorSubcoreMesh(core_axis_name="c", subcore_axis_name="s"),
    compiler_params=pltpu.CompilerParams(use_tc_tiling_on_sc=False),
    scratch_types=[
        pltpu.VMEM((2 * _WIN, _CHUNK), jnp.int32),
        pltpu.VMEM((2 * _WIN, _CHUNK), jnp.int32),
        pltpu.VMEM((_CHUNK, _DA), jnp.float32),
        pltpu.VMEM((_CHUNK, _DA), jnp.float32),
        pltpu.VMEM((_ROWS_PER_TILE, _DA), jnp.float32),
        pltpu.VMEM_SHARED((_ACC_ROWS, _DA), jnp.float32),
        pltpu.SemaphoreType.DMA,
        pltpu.SemaphoreType.DMA,
        pltpu.SemaphoreType.DMA,
    ],
)(_sc_aggregate_body)


_BLK = 128


def _neigh_mm_body(x_ref, w_ref, y0_ref, y1_ref):
    y = jnp.dot(x_ref[...], w_ref[...], preferred_element_type=jnp.float32)
    ones = jnp.ones((_BLK, _DA - _DH), jnp.float32)
    y0_ref[...] = jnp.concatenate([y[:, :_DH], ones], axis=1)
    y1_ref[...] = jnp.concatenate([y[:, _DH:], ones], axis=1)


def _neigh_mm(x, w_neigh_t):
    return pl.pallas_call(
        _neigh_mm_body,
        grid=(_N_PAD // _BLK,),
        in_specs=[pl.BlockSpec((_BLK, _D), lambda i: (i, 0)),
                  pl.BlockSpec((_D, _D), lambda i: (0, 0))],
        out_specs=[pl.BlockSpec((_BLK, _DA), lambda i: (i, 0)),
                   pl.BlockSpec((_BLK, _DA), lambda i: (i, 0))],
        out_shape=[jax.ShapeDtypeStruct((_N, _DA), jnp.float32),
                   jax.ShapeDtypeStruct((_N, _DA), jnp.float32)],
    )(x, w_neigh_t)


def _combine_body(x_ref, w_ref, b_ref, acc_ref, out_ref):
    a0 = acc_ref[0]
    a1 = acc_ref[1]
    deg = a0[:, _DH:_DH + 1]
    r = 1.0 / jnp.maximum(deg, 1.0)
    h = jnp.concatenate([a0[:, :_DH], a1[:, :_DH]], axis=1) * r
    out_ref[...] = (
        jnp.dot(x_ref[...], w_ref[...], preferred_element_type=jnp.float32)
        + h + b_ref[...]
    )


def _combine(x, w_self_t, b2d, acc):
    return pl.pallas_call(
        _combine_body,
        grid=(_N_PAD // _BLK,),
        in_specs=[pl.BlockSpec((_BLK, _D), lambda i: (i, 0)),
                  pl.BlockSpec((_D, _D), lambda i: (0, 0)),
                  pl.BlockSpec((1, _D), lambda i: (0, 0)),
                  pl.BlockSpec((_NC, _BLK, _DA), lambda i: (0, i, 0))],
        out_specs=pl.BlockSpec((_BLK, _D), lambda i: (i, 0)),
        out_shape=jax.ShapeDtypeStruct((_N, _D), jnp.float32),
    )(x, w_self_t, b2d, acc)


def kernel(x, edge_index, W_self, W_neigh, b):
    ei = edge_index.reshape(2, _ECHUNKS, _CHUNK)
    y0, y1 = _neigh_mm(x, W_neigh.T)
    acc = _sc_aggregate(y0, y1, ei)
    return _combine(x, W_self.T, b.reshape(1, _D), acc)


# speedup vs baseline: 10.5558x; 1.3597x over previous
import functools

import jax
import jax.numpy as jnp
from jax import lax
from jax.experimental import pallas as pl
from jax.experimental.pallas import tpu as pltpu
from jax.experimental.pallas import tpu_sc as plsc

_N = 10000
_E = 320000
_D = 128
_DH = _D // 2
_DA = _DH + 16

_NC = 2
_NS = 16

_CHUNK = 128
_CHUNKS_PER_T = 160
_WIN = 10
_ECHUNKS = _E // _CHUNK
_N_PAD = 10112
_ROWS_PER_TILE = _N_PAD // _NS
_ACC_ROWS = _N
_LAST_ROWS = _ACC_ROWS - 15 * _ROWS_PER_TILE
_FULL_CHUNKS = _CHUNKS_PER_T
_LAST_CHUNKS = _ECHUNKS - 15 * _CHUNKS_PER_T


def _sc_aggregate_body(y0_hbm, y1_hbm, ei_hbm, acc_out,
                       srcw_v, dstw_v, rows_v0, rows_v1,
                       stage_acc, acc_sh, gsem0, gsem1, wsem):
    c = lax.axis_index("c")
    s = lax.axis_index("s")
    base = s * _ROWS_PER_TILE
    cbase = s * _CHUNKS_PER_T

    z16 = jnp.zeros((16,), jnp.float32)

    def _zrow(i, carry):
        for k in range(_DA // 16):
            stage_acc[i, pl.ds(k * 16, 16)] = z16
        return carry

    lax.fori_loop(0, _ROWS_PER_TILE, _zrow, 0)

    @pl.when(s < 15)
    def _():
        pltpu.sync_copy(stage_acc, acc_sh.at[pl.ds(base, _ROWS_PER_TILE)])

    @pl.when(s == 15)
    def _():
        pltpu.sync_copy(stage_acc.at[pl.ds(0, _LAST_ROWS)],
                        acc_sh.at[pl.ds(15 * _ROWS_PER_TILE, _LAST_ROWS)])

    plsc.subcore_barrier()

    def _run(y_hbm):
        rows = (rows_v0, rows_v1)
        sems = (gsem0, gsem1)

        def _stage(w, q):
            pltpu.async_copy(ei_hbm.at[0, pl.ds(cbase + w * _WIN, _WIN)],
                             srcw_v.at[pl.ds(q * _WIN, _WIN)], wsem)
            pltpu.async_copy(ei_hbm.at[1, pl.ds(cbase + w * _WIN, _WIN)],
                             dstw_v.at[pl.ds(q * _WIN, _WIN)], wsem)

        def _drain_stage():
            pltpu.make_async_copy(ei_hbm.at[0, pl.ds(0, _WIN)],
                                  srcw_v.at[pl.ds(0, _WIN)], wsem).wait()
            pltpu.make_async_copy(ei_hbm.at[1, pl.ds(0, _WIN)],
                                  dstw_v.at[pl.ds(0, _WIN)], wsem).wait()

        def _start(r, b):
            pltpu.async_copy(y_hbm.at[srcw_v.at[r]], rows[b], sems[b])

        def _scatter(r, b):
            pltpu.make_async_copy(y_hbm.at[srcw_v.at[0]], rows[b],
                                  sems[b]).wait()
            pltpu.sync_copy(rows[b], acc_sh.at[dstw_v.at[r]], add=True)

        nwin = jnp.where(s == 15, _LAST_CHUNKS // _WIN, _FULL_CHUNKS // _WIN)
        _stage(0, 0)
        _drain_stage()

        def win(w, carry):
            q = lax.rem(w, 2)
            rbase = q * _WIN

            @pl.when(w + 1 < nwin)
            def _():
                _stage(w + 1, 1 - q)

            _start(rbase, 0)

            def pair(p, carry2):
                r0 = rbase + 2 * p
                _start(r0 + 1, 1)
                _scatter(r0, 0)

                @pl.when(p + 1 < _WIN // 2)
                def _():
                    _start(r0 + 2, 0)

                _scatter(r0 + 1, 1)
                return carry2

            lax.fori_loop(0, _WIN // 2, pair, 0)

            @pl.when(w + 1 < nwin)
            def _():
                _drain_stage()

            return carry

        lax.fori_loop(0, nwin, win, 0)

    @pl.when(c == 0)
    def _():
        _run(y0_hbm)

    @pl.when(c == 1)
    def _():
        _run(y1_hbm)

    plsc.subcore_barrier()

    @pl.when(s < 15)
    def _():
        pltpu.sync_copy(acc_sh.at[pl.ds(base, _ROWS_PER_TILE)], stage_acc)
        pltpu.sync_copy(stage_acc, acc_out.at[c, pl.ds(base, _ROWS_PER_TILE)])

    @pl.when(s == 15)
    def _():
        pltpu.sync_copy(acc_sh.at[pl.ds(15 * _ROWS_PER_TILE, _LAST_ROWS)],
                        stage_acc.at[pl.ds(0, _LAST_ROWS)])
        pltpu.sync_copy(stage_acc.at[pl.ds(0, _LAST_ROWS)],
                        acc_out.at[c, pl.ds(15 * _ROWS_PER_TILE, _LAST_ROWS)])


_sc_aggregate = functools.partial(
    pl.kernel,
    out_type=jax.ShapeDtypeStruct((_NC, _ACC_ROWS, _DA), jnp.float32),
    mesh=plsc.VectorSubcoreMesh(core_axis_name="c", subcore_axis_name="s"),
    compiler_params=pltpu.CompilerParams(use_tc_tiling_on_sc=False),
    scratch_types=[
        pltpu.VMEM((2 * _WIN, _CHUNK), jnp.int32),
        pltpu.VMEM((2 * _WIN, _CHUNK), jnp.int32),
        pltpu.VMEM((_CHUNK, _DA), jnp.float32),
        pltpu.VMEM((_CHUNK, _DA), jnp.float32),
        pltpu.VMEM((_ROWS_PER_TILE, _DA), jnp.float32),
        pltpu.VMEM_SHARED((_ACC_ROWS, _DA), jnp.float32),
        pltpu.SemaphoreType.DMA,
        pltpu.SemaphoreType.DMA,
        pltpu.SemaphoreType.DMA,
    ],
)(_sc_aggregate_body)


def _neigh_mm_body(x_ref, w_ref, y0_ref, y1_ref):
    y = jnp.dot(x_ref[...], w_ref[...], preferred_element_type=jnp.float32)
    ones = jnp.ones((_N, _DA - _DH), jnp.float32)
    y0_ref[...] = jnp.concatenate([y[:, :_DH], ones], axis=1)
    y1_ref[...] = jnp.concatenate([y[:, _DH:], ones], axis=1)


def _neigh_mm(x, w_neigh_t):
    return pl.pallas_call(
        _neigh_mm_body,
        out_shape=[jax.ShapeDtypeStruct((_N, _DA), jnp.float32),
                   jax.ShapeDtypeStruct((_N, _DA), jnp.float32)],
    )(x, w_neigh_t)


def _combine_body(x_ref, w_ref, b_ref, acc_ref, out_ref):
    a0 = acc_ref[0]
    a1 = acc_ref[1]
    deg = a0[:, _DH:_DH + 1]
    r = 1.0 / jnp.maximum(deg, 1.0)
    h = jnp.concatenate([a0[:, :_DH], a1[:, :_DH]], axis=1) * r
    out_ref[...] = (
        jnp.dot(x_ref[...], w_ref[...], preferred_element_type=jnp.float32)
        + h + b_ref[...]
    )


def _combine(x, w_self_t, b2d, acc):
    return pl.pallas_call(
        _combine_body,
        out_shape=jax.ShapeDtypeStruct((_N, _D), jnp.float32),
    )(x, w_self_t, b2d, acc)


def kernel(x, edge_index, W_self, W_neigh, b):
    ei = edge_index.reshape(2, _ECHUNKS, _CHUNK)
    y0, y1 = _neigh_mm(x, W_neigh.T)
    acc = _sc_aggregate(y0, y1, ei)
    return _combine(x, W_self.T, b.reshape(1, _D), acc)


# speedup vs baseline: 10.8473x; 1.0276x over previous
import functools

import jax
import jax.numpy as jnp
from jax import lax
from jax.experimental import pallas as pl
from jax.experimental.pallas import tpu as pltpu
from jax.experimental.pallas import tpu_sc as plsc

_N = 10000
_E = 320000
_D = 128
_DH = _D // 2
_DA = _DH + 8

_NC = 2
_NS = 16

_CHUNK = 128
_CHUNKS_PER_T = 160
_WIN = 10
_ECHUNKS = _E // _CHUNK
_N_PAD = 10112
_ROWS_PER_TILE = _N_PAD // _NS
_ACC_ROWS = _N
_LAST_ROWS = _ACC_ROWS - 15 * _ROWS_PER_TILE
_FULL_CHUNKS = _CHUNKS_PER_T
_LAST_CHUNKS = _ECHUNKS - 15 * _CHUNKS_PER_T


def _sc_aggregate_body(y0_hbm, y1_hbm, ei_hbm, acc_out,
                       srcw_v, dstw_v, rows_v0, rows_v1,
                       stage_acc, acc_sh, gsem0, gsem1, wsem):
    c = lax.axis_index("c")
    s = lax.axis_index("s")
    base = s * _ROWS_PER_TILE
    cbase = s * _CHUNKS_PER_T

    z16 = jnp.zeros((16,), jnp.float32)

    def _zrow(i, carry):
        for k in range(_DA // 16):
            stage_acc[i, pl.ds(k * 16, 16)] = z16
        if _DA % 16:
            stage_acc[i, pl.ds(_DA - 16, 16)] = z16
        return carry

    lax.fori_loop(0, _ROWS_PER_TILE, _zrow, 0)

    @pl.when(s < 15)
    def _():
        pltpu.sync_copy(stage_acc, acc_sh.at[pl.ds(base, _ROWS_PER_TILE)])

    @pl.when(s == 15)
    def _():
        pltpu.sync_copy(stage_acc.at[pl.ds(0, _LAST_ROWS)],
                        acc_sh.at[pl.ds(15 * _ROWS_PER_TILE, _LAST_ROWS)])

    plsc.subcore_barrier()

    def _run(y_hbm):
        rows = (rows_v0, rows_v1)
        sems = (gsem0, gsem1)

        def _stage(w, q):
            pltpu.async_copy(ei_hbm.at[0, pl.ds(cbase + w * _WIN, _WIN)],
                             srcw_v.at[pl.ds(q * _WIN, _WIN)], wsem)
            pltpu.async_copy(ei_hbm.at[1, pl.ds(cbase + w * _WIN, _WIN)],
                             dstw_v.at[pl.ds(q * _WIN, _WIN)], wsem)

        def _drain_stage():
            pltpu.make_async_copy(ei_hbm.at[0, pl.ds(0, _WIN)],
                                  srcw_v.at[pl.ds(0, _WIN)], wsem).wait()
            pltpu.make_async_copy(ei_hbm.at[1, pl.ds(0, _WIN)],
                                  dstw_v.at[pl.ds(0, _WIN)], wsem).wait()

        def _start(r, b):
            pltpu.async_copy(y_hbm.at[srcw_v.at[r]], rows[b], sems[b])

        def _scatter(r, b):
            pltpu.make_async_copy(y_hbm.at[srcw_v.at[0]], rows[b],
                                  sems[b]).wait()
            pltpu.sync_copy(rows[b], acc_sh.at[dstw_v.at[r]], add=True)

        nwin = jnp.where(s == 15, _LAST_CHUNKS // _WIN, _FULL_CHUNKS // _WIN)
        _stage(0, 0)
        _drain_stage()

        def win(w, carry):
            q = lax.rem(w, 2)
            rbase = q * _WIN

            @pl.when(w + 1 < nwin)
            def _():
                _stage(w + 1, 1 - q)

            _start(rbase, 0)

            def pair(p, carry2):
                r0 = rbase + 2 * p
                _start(r0 + 1, 1)
                _scatter(r0, 0)

                @pl.when(p + 1 < _WIN // 2)
                def _():
                    _start(r0 + 2, 0)

                _scatter(r0 + 1, 1)
                return carry2

            lax.fori_loop(0, _WIN // 2, pair, 0)

            @pl.when(w + 1 < nwin)
            def _():
                _drain_stage()

            return carry

        lax.fori_loop(0, nwin, win, 0)

    @pl.when(c == 0)
    def _():
        _run(y0_hbm)

    @pl.when(c == 1)
    def _():
        _run(y1_hbm)

    plsc.subcore_barrier()

    @pl.when(s < 15)
    def _():
        pltpu.sync_copy(acc_sh.at[pl.ds(base, _ROWS_PER_TILE)], stage_acc)
        pltpu.sync_copy(stage_acc, acc_out.at[c, pl.ds(base, _ROWS_PER_TILE)])

    @pl.when(s == 15)
    def _():
        pltpu.sync_copy(acc_sh.at[pl.ds(15 * _ROWS_PER_TILE, _LAST_ROWS)],
                        stage_acc.at[pl.ds(0, _LAST_ROWS)])
        pltpu.sync_copy(stage_acc.at[pl.ds(0, _LAST_ROWS)],
                        acc_out.at[c, pl.ds(15 * _ROWS_PER_TILE, _LAST_ROWS)])


_sc_aggregate = functools.partial(
    pl.kernel,
    out_type=jax.ShapeDtypeStruct((_NC, _ACC_ROWS, _DA), jnp.float32),
    mesh=plsc.VectorSubcoreMesh(core_axis_name="c", subcore_axis_name="s"),
    compiler_params=pltpu.CompilerParams(use_tc_tiling_on_sc=False),
    scratch_types=[
        pltpu.VMEM((2 * _WIN, _CHUNK), jnp.int32),
        pltpu.VMEM((2 * _WIN, _CHUNK), jnp.int32),
        pltpu.VMEM((_CHUNK, _DA), jnp.float32),
        pltpu.VMEM((_CHUNK, _DA), jnp.float32),
        pltpu.VMEM((_ROWS_PER_TILE, _DA), jnp.float32),
        pltpu.VMEM_SHARED((_ACC_ROWS, _DA), jnp.float32),
        pltpu.SemaphoreType.DMA,
        pltpu.SemaphoreType.DMA,
        pltpu.SemaphoreType.DMA,
    ],
)(_sc_aggregate_body)


def _neigh_mm_body(x_ref, w_ref, y0_ref, y1_ref):
    y = jnp.dot(x_ref[...], w_ref[...], preferred_element_type=jnp.float32)
    ones = jnp.ones((_N, _DA - _DH), jnp.float32)
    y0_ref[...] = jnp.concatenate([y[:, :_DH], ones], axis=1)
    y1_ref[...] = jnp.concatenate([y[:, _DH:], ones], axis=1)


def _neigh_mm(x, w_neigh_t):
    return pl.pallas_call(
        _neigh_mm_body,
        out_shape=[jax.ShapeDtypeStruct((_N, _DA), jnp.float32),
                   jax.ShapeDtypeStruct((_N, _DA), jnp.float32)],
    )(x, w_neigh_t)


def _combine_body(x_ref, w_ref, b_ref, acc_ref, out_ref):
    a0 = acc_ref[0]
    a1 = acc_ref[1]
    deg = a0[:, _DH:_DH + 1]
    r = 1.0 / jnp.maximum(deg, 1.0)
    h = jnp.concatenate([a0[:, :_DH], a1[:, :_DH]], axis=1) * r
    out_ref[...] = (
        jnp.dot(x_ref[...], w_ref[...], preferred_element_type=jnp.float32)
        + h + b_ref[...]
    )


def _combine(x, w_self_t, b2d, acc):
    return pl.pallas_call(
        _combine_body,
        out_shape=jax.ShapeDtypeStruct((_N, _D), jnp.float32),
    )(x, w_self_t, b2d, acc)


def kernel(x, edge_index, W_self, W_neigh, b):
    ei = edge_index.reshape(2, _ECHUNKS, _CHUNK)
    y0, y1 = _neigh_mm(x, W_neigh.T)
    acc = _sc_aggregate(y0, y1, ei)
    return _combine(x, W_self.T, b.reshape(1, _D), acc)
